# baseline (speedup 1.0000x reference)
import functools

import jax
import jax.numpy as jnp
from jax.experimental import pallas as pl
from jax.experimental.pallas import tpu as pltpu

_VMEM_LIMIT = 48 * 1024 * 1024
_EPS = 1e-5


def _round_up(a, b):
    return (a + b - 1) // b * b


def _scale_shift(st_ref, g_ref, be_ref, m):
    st = jnp.sum(st_ref[...], axis=0)
    mean = st[0:1] / m
    var = jnp.maximum(st[1:2] / m - mean * mean, 0.0)
    scale = g_ref[...] / jnp.sqrt(var + _EPS)
    shift = be_ref[...] - mean * scale
    return scale, shift


def _k1_conv1_ds(x_ref, w1_ref, b1_ref, wd_ref, bd_ref,
                 y1_ref, st1_ref, yd_ref, std_ref, *, hh, w, wo):
    xb = x_ref[0].astype(jnp.bfloat16)
    cin = xb.shape[1]
    y1 = jnp.dot(xb, w1_ref[...].astype(jnp.bfloat16),
                 preferred_element_type=jnp.float32)
    y1 = y1 + b1_ref[...]
    y1_ref[0] = y1.astype(jnp.bfloat16)
    st1_ref[0, 0:1, :] = jnp.sum(y1, axis=0, keepdims=True)
    st1_ref[0, 1:2, :] = jnp.sum(y1 * y1, axis=0, keepdims=True)

    xe = xb.reshape(hh // 2, 2, w, cin)[:, 0]
    xe = xe.reshape((hh // 2) * wo, 2 * cin)
    wdx = jnp.pad(wd_ref[...].astype(jnp.bfloat16), ((0, cin), (0, 0)))
    yd = jnp.dot(xe, wdx, preferred_element_type=jnp.float32) + bd_ref[...]
    yd_ref[...] = yd.astype(jnp.bfloat16)
    std_ref[0, 0:1, :] = jnp.sum(yd, axis=0, keepdims=True)
    std_ref[0, 1:2, :] = jnp.sum(yd * yd, axis=0, keepdims=True)


def _k2_conv2(y1_ref, st1_ref, g1_ref, be1_ref, w2_ref, b2_ref,
              y2_ref, st2_ref, *, m1, h, w, ho, wo, wo_pad, hp, wp):
    c = w2_ref.shape[1]
    s1, h1 = _scale_shift(st1_ref, g1_ref, be1_ref, m1)
    a = y1_ref[0].astype(jnp.float32) * s1 + h1
    a = jnp.where(a >= 0, a, 0.02 * a).astype(jnp.bfloat16)
    ap = jnp.pad(a, ((1, hp - h - 1), (1, wp - w - 1), (0, 0)))
    hs = ap.reshape(hp // 2, 2, wp, c)
    merged = [hs[:, r].reshape(hp // 2, wp // 2, 2 * c) for r in range(2)]
    wins = []
    for di in range(3):
        for dj in range(3):
            r, s = di % 2, dj % 2
            v = merged[r][:, :, s * c:(s + 1) * c]
            if dj == 2:
                v = jnp.roll(v, -1, axis=1)
            wins.append(v[di // 2:di // 2 + ho].reshape(ho * (wp // 2), c))
    xw = jnp.concatenate(wins, axis=1)
    y2 = jnp.dot(xw, w2_ref[...].astype(jnp.bfloat16),
                 preferred_element_type=jnp.float32) + b2_ref[...]
    y2 = y2.reshape(ho, wp // 2, c)[:, :wo, :].reshape(ho * wo, c)
    y2_ref[...] = y2.astype(jnp.bfloat16)
    st2_ref[0, 0:1, :] = jnp.sum(y2, axis=0, keepdims=True)
    st2_ref[0, 1:2, :] = jnp.sum(y2 * y2, axis=0, keepdims=True)


def _k3_conv3(y2_ref, st2_ref, g2_ref, be2_ref, w3_ref, b3_ref,
              y3_ref, st3_ref, *, m2):
    s2, h2 = _scale_shift(st2_ref, g2_ref, be2_ref, m2)
    t = y2_ref[...].astype(jnp.float32) * s2 + h2
    a2 = jnp.where(t >= 0, t, 0.02 * t).astype(jnp.bfloat16)
    y3 = jnp.dot(a2, w3_ref[...].astype(jnp.bfloat16),
                 preferred_element_type=jnp.float32) + b3_ref[...]
    y3_ref[...] = y3.astype(jnp.bfloat16)
    st3_ref[0, 0:1, :] = jnp.sum(y3, axis=0, keepdims=True)
    st3_ref[0, 1:2, :] = jnp.sum(y3 * y3, axis=0, keepdims=True)


def _k4_residual(y3_ref, st3_ref, g3_ref, be3_ref, yd_ref, std_ref,
                 gd_ref, bed_ref, o_ref, *, m2):
    s3, h3 = _scale_shift(st3_ref, g3_ref, be3_ref, m2)
    sd, hd = _scale_shift(std_ref, gd_ref, bed_ref, m2)
    tm, c4 = o_ref.shape
    ydv = yd_ref[...].reshape(tm, c4)
    z = y3_ref[...].astype(jnp.float32) * s3 + h3
    sk = ydv.astype(jnp.float32) * sd + hd
    y = z + sk
    o_ref[...] = jnp.where(y >= 0, y, 0.01 * y)


def kernel(x, w1, b1, g1, be1, w2, b2, g2, be2, w3, b3, g3, be3,
           wd, bd, gd, bed):
    n, cin, h, w = x.shape
    cout = w1.shape[1]
    c4 = w3.shape[1]
    ho = (h + 2 - 3) // 2 + 1
    wo = (w + 2 - 3) // 2 + 1
    wo_pad = _round_up(wo, 8)
    hw = h * w
    sp = ho * wo
    bf = jnp.bfloat16
    hp = _round_up(h + 2, 16)
    wp = _round_up(w + 2, 16)

    x_img = jnp.transpose(x, (0, 2, 3, 1)).reshape(n, hw, cin)

    m1 = n * hw
    nh = 2 if (h // 2) % 4 == 0 else 1
    hh = h // nh
    spw = (hh // 2) * wo
    y1, st1, yd, std = pl.pallas_call(
        functools.partial(_k1_conv1_ds, hh=hh, w=w, wo=wo),
        out_shape=(jax.ShapeDtypeStruct((n * nh, hw // nh, cout), bf),
                   jax.ShapeDtypeStruct((n * nh, 2, cout), jnp.float32),
                   jax.ShapeDtypeStruct((sp, n * c4), bf),
                   jax.ShapeDtypeStruct((n * nh, 2, c4), jnp.float32)),
        grid=(n * nh,),
        in_specs=[pl.BlockSpec((1, hw // nh, cin), lambda i: (i, 0, 0)),
                  pl.BlockSpec((cin, cout), lambda i: (0, 0)),
                  pl.BlockSpec((1, cout), lambda i: (0, 0)),
                  pl.BlockSpec((cin, c4), lambda i: (0, 0)),
                  pl.BlockSpec((1, c4), lambda i: (0, 0))],
        out_specs=(pl.BlockSpec((1, hw // nh, cout), lambda i: (i, 0, 0)),
                   pl.BlockSpec((1, 2, cout), lambda i: (i, 0, 0)),
                   pl.BlockSpec((spw, c4),
                                lambda i: (i % nh, i // nh) if nh == 2
                                else (0, i)),
                   pl.BlockSpec((1, 2, c4), lambda i: (i, 0, 0))),
        compiler_params=pltpu.CompilerParams(
            dimension_semantics=("parallel",),
            vmem_limit_bytes=_VMEM_LIMIT),
    )(x_img.reshape(n * nh, hw // nh, cin), w1, b1.reshape(1, cout), wd,
      bd.reshape(1, c4))

    y2, st2 = pl.pallas_call(
        functools.partial(_k2_conv2, m1=m1, h=h, w=w, ho=ho, wo=wo,
                          wo_pad=wo_pad, hp=hp, wp=wp),
        out_shape=(jax.ShapeDtypeStruct((sp, n * cout), bf),
                   jax.ShapeDtypeStruct((n, 2, cout), jnp.float32)),
        grid=(n,),
        in_specs=[pl.BlockSpec((1, h, w, cout), lambda i: (i, 0, 0, 0)),
                  pl.BlockSpec((n * nh, 2, cout), lambda i: (0, 0, 0)),
                  pl.BlockSpec((1, cout), lambda i: (0, 0)),
                  pl.BlockSpec((1, cout), lambda i: (0, 0)),
                  pl.BlockSpec((9 * cout, cout), lambda i: (0, 0)),
                  pl.BlockSpec((1, cout), lambda i: (0, 0))],
        out_specs=(pl.BlockSpec((sp, cout), lambda i: (0, i)),
                   pl.BlockSpec((1, 2, cout), lambda i: (i, 0, 0))),
        compiler_params=pltpu.CompilerParams(
            dimension_semantics=("parallel",),
            vmem_limit_bytes=_VMEM_LIMIT),
    )(y1.reshape(n, h, w, cout), st1, g1.reshape(1, cout),
      be1.reshape(1, cout), w2.reshape(9 * cout, cout), b2.reshape(1, cout))

    m2 = n * sp

    tm = sp
    nt = m2 // tm
    y3, st3 = pl.pallas_call(
        functools.partial(_k3_conv3, m2=m2),
        out_shape=(jax.ShapeDtypeStruct((m2, c4), bf),
                   jax.ShapeDtypeStruct((nt, 2, c4), jnp.float32)),
        grid=(nt,),
        in_specs=[pl.BlockSpec((tm, cout), lambda i: (i, 0)),
                  pl.BlockSpec((n, 2, cout), lambda i: (0, 0, 0)),
                  pl.BlockSpec((1, cout), lambda i: (0, 0)),
                  pl.BlockSpec((1, cout), lambda i: (0, 0)),
                  pl.BlockSpec((cout, c4), lambda i: (0, 0)),
                  pl.BlockSpec((1, c4), lambda i: (0, 0))],
        out_specs=(pl.BlockSpec((tm, c4), lambda i: (i, 0)),
                   pl.BlockSpec((1, 2, c4), lambda i: (i, 0, 0))),
        compiler_params=pltpu.CompilerParams(
            dimension_semantics=("parallel",),
            vmem_limit_bytes=_VMEM_LIMIT),
    )(y2.reshape(m2, cout), st2, g2.reshape(1, cout), be2.reshape(1, cout),
      w3, b3.reshape(1, c4))

    nt4 = 7 if (sp % 7 == 0 and (sp // 7) % 8 == 0) else 1
    tm4 = m2 // nt4
    out = pl.pallas_call(
        functools.partial(_k4_residual, m2=m2),
        out_shape=jax.ShapeDtypeStruct((m2, c4), jnp.float32),
        grid=(nt4,),
        in_specs=[pl.BlockSpec((tm4, c4), lambda i: (i, 0)),
                  pl.BlockSpec((nt, 2, c4), lambda i: (0, 0, 0)),
                  pl.BlockSpec((1, c4), lambda i: (0, 0)),
                  pl.BlockSpec((1, c4), lambda i: (0, 0)),
                  pl.BlockSpec((tm4 // n, n * c4), lambda i: (i, 0)),
                  pl.BlockSpec((n * nh, 2, c4), lambda i: (0, 0, 0)),
                  pl.BlockSpec((1, c4), lambda i: (0, 0)),
                  pl.BlockSpec((1, c4), lambda i: (0, 0))],
        out_specs=pl.BlockSpec((tm4, c4), lambda i: (i, 0)),
        compiler_params=pltpu.CompilerParams(
            dimension_semantics=("parallel",),
            vmem_limit_bytes=_VMEM_LIMIT),
    )(y3, st3, g3.reshape(1, c4), be3.reshape(1, c4),
      yd, std, gd.reshape(1, c4), bed.reshape(1, c4))

    return jnp.transpose(out.reshape(ho, wo, n, c4), (2, 3, 0, 1))


# speedup vs baseline: 1.1653x; 1.1653x over previous
import functools

import jax
import jax.numpy as jnp
from jax.experimental import pallas as pl
from jax.experimental.pallas import tpu as pltpu

_VMEM_LIMIT = 48 * 1024 * 1024
_EPS = 1e-5


def _round_up(a, b):
    return (a + b - 1) // b * b


def _scale_shift(st_ref, g_ref, be_ref, m):
    st = jnp.sum(st_ref[...], axis=0)
    mean = st[0:1] / m
    var = jnp.maximum(st[1:2] / m - mean * mean, 0.0)
    scale = g_ref[...] / jnp.sqrt(var + _EPS)
    shift = be_ref[...] - mean * scale
    return scale, shift


def _k1_conv1_ds(x_ref, w1_ref, b1_ref, wd_ref, bd_ref,
                 y1_ref, st1_ref, yd_ref, std_ref, *, h, w, ho, wo):
    xb = x_ref[0].astype(jnp.bfloat16)
    cin = xb.shape[1]
    y1 = jnp.dot(xb, w1_ref[...].astype(jnp.bfloat16),
                 preferred_element_type=jnp.float32)
    y1 = y1 + b1_ref[...]
    y1_ref[0] = y1.astype(jnp.bfloat16)
    st1_ref[0, 0:1, :] = jnp.sum(y1, axis=0, keepdims=True)
    st1_ref[0, 1:2, :] = jnp.sum(y1 * y1, axis=0, keepdims=True)

    xe = xb.reshape(h // 2, 2, w, cin)[:, 0]
    xe = xe.reshape(ho, wo, 2 * cin)
    xe = xe.reshape(ho * wo, 2 * cin)
    wdx = jnp.pad(wd_ref[...].astype(jnp.bfloat16), ((0, cin), (0, 0)))
    yd = jnp.dot(xe, wdx, preferred_element_type=jnp.float32) + bd_ref[...]
    yd_ref[...] = yd.astype(jnp.bfloat16)
    std_ref[0, 0:1, :] = jnp.sum(yd, axis=0, keepdims=True)
    std_ref[0, 1:2, :] = jnp.sum(yd * yd, axis=0, keepdims=True)


def _k2_conv2(y1_ref, st1_ref, g1_ref, be1_ref, w2_ref, b2_ref,
              y2_ref, st2_ref, *, m1, h, w, ho, wo, wo_pad, hp, wp):
    c = w2_ref.shape[1]
    s1, h1 = _scale_shift(st1_ref, g1_ref, be1_ref, m1)
    a = y1_ref[0].astype(jnp.float32) * s1 + h1
    a = jnp.where(a >= 0, a, 0.02 * a).astype(jnp.bfloat16)
    ap = jnp.pad(a, ((1, hp - h - 1), (1, wp - w - 1), (0, 0)))
    hs = ap.reshape(hp // 2, 2, wp, c)
    merged = [hs[:, r].reshape(hp // 2, wp // 2, 2 * c) for r in range(2)]
    wins = []
    for di in range(3):
        for dj in range(3):
            r, s = di % 2, dj % 2
            v = merged[r][:, :, s * c:(s + 1) * c]
            if dj == 2:
                v = jnp.roll(v, -1, axis=1)
            wins.append(v[di // 2:di // 2 + ho].reshape(ho * (wp // 2), c))
    xw = jnp.concatenate(wins, axis=1)
    y2 = jnp.dot(xw, w2_ref[...].astype(jnp.bfloat16),
                 preferred_element_type=jnp.float32) + b2_ref[...]
    y2 = y2.reshape(ho, wp // 2, c)[:, :wo, :].reshape(ho * wo, c)
    y2_ref[...] = y2.astype(jnp.bfloat16)
    st2_ref[0, 0:1, :] = jnp.sum(y2, axis=0, keepdims=True)
    st2_ref[0, 1:2, :] = jnp.sum(y2 * y2, axis=0, keepdims=True)


def _conv3_rows(y2v, st2_ref, g2_ref, be2_ref, w3_ref, b3_ref, m2):
    s2, h2 = _scale_shift(st2_ref, g2_ref, be2_ref, m2)
    t = y2v.astype(jnp.float32) * s2 + h2
    a2 = jnp.where(t >= 0, t, 0.02 * t).astype(jnp.bfloat16)
    return jnp.dot(a2, w3_ref[...].astype(jnp.bfloat16),
                   preferred_element_type=jnp.float32) + b3_ref[...]


def _k3_stats(y2_ref, st2_ref, g2_ref, be2_ref, w3_ref, b3_ref,
              st3_ref, *, m2):
    y3 = _conv3_rows(y2_ref[...], st2_ref, g2_ref, be2_ref, w3_ref, b3_ref,
                     m2)
    st3_ref[0, 0:1, :] = jnp.sum(y3, axis=0, keepdims=True)
    st3_ref[0, 1:2, :] = jnp.sum(y3 * y3, axis=0, keepdims=True)


def _k4_residual(y2_ref, st2_ref, g2_ref, be2_ref, w3_ref, b3_ref,
                 st3_ref, g3_ref, be3_ref, yd_ref, std_ref,
                 gd_ref, bed_ref, o_ref, *, m2, cout):
    tm, c4 = o_ref.shape
    y3 = _conv3_rows(y2_ref[...].reshape(tm, cout), st2_ref, g2_ref,
                     be2_ref, w3_ref, b3_ref, m2)
    s3, h3 = _scale_shift(st3_ref, g3_ref, be3_ref, m2)
    sd, hd = _scale_shift(std_ref, gd_ref, bed_ref, m2)
    ydv = yd_ref[...].reshape(tm, c4)
    z = y3 * s3 + h3
    sk = ydv.astype(jnp.float32) * sd + hd
    y = z + sk
    o_ref[...] = jnp.where(y >= 0, y, 0.01 * y)


def kernel(x, w1, b1, g1, be1, w2, b2, g2, be2, w3, b3, g3, be3,
           wd, bd, gd, bed):
    n, cin, h, w = x.shape
    cout = w1.shape[1]
    c4 = w3.shape[1]
    ho = (h + 2 - 3) // 2 + 1
    wo = (w + 2 - 3) // 2 + 1
    wo_pad = _round_up(wo, 8)
    hw = h * w
    sp = ho * wo
    bf = jnp.bfloat16
    hp = _round_up(h + 2, 16)
    wp = _round_up(w + 2, 16)

    x_img = jnp.transpose(x, (0, 2, 3, 1)).reshape(n, hw, cin)

    m1 = n * hw
    y1, st1, yd, std = pl.pallas_call(
        functools.partial(_k1_conv1_ds, h=h, w=w, ho=ho, wo=wo),
        out_shape=(jax.ShapeDtypeStruct((n, hw, cout), bf),
                   jax.ShapeDtypeStruct((n, 2, cout), jnp.float32),
                   jax.ShapeDtypeStruct((sp, n * c4), bf),
                   jax.ShapeDtypeStruct((n, 2, c4), jnp.float32)),
        grid=(n,),
        in_specs=[pl.BlockSpec((1, hw, cin), lambda i: (i, 0, 0)),
                  pl.BlockSpec((cin, cout), lambda i: (0, 0)),
                  pl.BlockSpec((1, cout), lambda i: (0, 0)),
                  pl.BlockSpec((cin, c4), lambda i: (0, 0)),
                  pl.BlockSpec((1, c4), lambda i: (0, 0))],
        out_specs=(pl.BlockSpec((1, hw, cout), lambda i: (i, 0, 0)),
                   pl.BlockSpec((1, 2, cout), lambda i: (i, 0, 0)),
                   pl.BlockSpec((sp, c4), lambda i: (0, i)),
                   pl.BlockSpec((1, 2, c4), lambda i: (i, 0, 0))),
        compiler_params=pltpu.CompilerParams(
            dimension_semantics=("parallel",),
            vmem_limit_bytes=_VMEM_LIMIT),
    )(x_img, w1, b1.reshape(1, cout), wd, bd.reshape(1, c4))

    y2, st2 = pl.pallas_call(
        functools.partial(_k2_conv2, m1=m1, h=h, w=w, ho=ho, wo=wo,
                          wo_pad=wo_pad, hp=hp, wp=wp),
        out_shape=(jax.ShapeDtypeStruct((sp, n * cout), bf),
                   jax.ShapeDtypeStruct((n, 2, cout), jnp.float32)),
        grid=(n,),
        in_specs=[pl.BlockSpec((1, h, w, cout), lambda i: (i, 0, 0, 0)),
                  pl.BlockSpec((n, 2, cout), lambda i: (0, 0, 0)),
                  pl.BlockSpec((1, cout), lambda i: (0, 0)),
                  pl.BlockSpec((1, cout), lambda i: (0, 0)),
                  pl.BlockSpec((9 * cout, cout), lambda i: (0, 0)),
                  pl.BlockSpec((1, cout), lambda i: (0, 0))],
        out_specs=(pl.BlockSpec((sp, cout), lambda i: (0, i)),
                   pl.BlockSpec((1, 2, cout), lambda i: (i, 0, 0))),
        compiler_params=pltpu.CompilerParams(
            dimension_semantics=("parallel",),
            vmem_limit_bytes=_VMEM_LIMIT),
    )(y1.reshape(n, h, w, cout), st1, g1.reshape(1, cout),
      be1.reshape(1, cout), w2.reshape(9 * cout, cout), b2.reshape(1, cout))

    m2 = n * sp

    st3 = pl.pallas_call(
        functools.partial(_k3_stats, m2=m2),
        out_shape=jax.ShapeDtypeStruct((n, 2, c4), jnp.float32),
        grid=(n,),
        in_specs=[pl.BlockSpec((sp, cout), lambda i: (0, i)),
                  pl.BlockSpec((n, 2, cout), lambda i: (0, 0, 0)),
                  pl.BlockSpec((1, cout), lambda i: (0, 0)),
                  pl.BlockSpec((1, cout), lambda i: (0, 0)),
                  pl.BlockSpec((cout, c4), lambda i: (0, 0)),
                  pl.BlockSpec((1, c4), lambda i: (0, 0))],
        out_specs=pl.BlockSpec((1, 2, c4), lambda i: (i, 0, 0)),
        compiler_params=pltpu.CompilerParams(
            dimension_semantics=("parallel",),
            vmem_limit_bytes=_VMEM_LIMIT),
    )(y2, st2, g2.reshape(1, cout), be2.reshape(1, cout),
      w3, b3.reshape(1, c4))

    nt4 = 7 if (sp % 7 == 0 and (sp // 7) % 8 == 0) else 1
    tm4 = m2 // nt4
    out = pl.pallas_call(
        functools.partial(_k4_residual, m2=m2, cout=cout),
        out_shape=jax.ShapeDtypeStruct((m2, c4), jnp.float32),
        grid=(nt4,),
        in_specs=[pl.BlockSpec((tm4 // n, n * cout), lambda i: (i, 0)),
                  pl.BlockSpec((n, 2, cout), lambda i: (0, 0, 0)),
                  pl.BlockSpec((1, cout), lambda i: (0, 0)),
                  pl.BlockSpec((1, cout), lambda i: (0, 0)),
                  pl.BlockSpec((cout, c4), lambda i: (0, 0)),
                  pl.BlockSpec((1, c4), lambda i: (0, 0)),
                  pl.BlockSpec((n, 2, c4), lambda i: (0, 0, 0)),
                  pl.BlockSpec((1, c4), lambda i: (0, 0)),
                  pl.BlockSpec((1, c4), lambda i: (0, 0)),
                  pl.BlockSpec((tm4 // n, n * c4), lambda i: (i, 0)),
                  pl.BlockSpec((n, 2, c4), lambda i: (0, 0, 0)),
                  pl.BlockSpec((1, c4), lambda i: (0, 0)),
                  pl.BlockSpec((1, c4), lambda i: (0, 0))],
        out_specs=pl.BlockSpec((tm4, c4), lambda i: (i, 0)),
        compiler_params=pltpu.CompilerParams(
            dimension_semantics=("parallel",),
            vmem_limit_bytes=_VMEM_LIMIT),
    )(y2, st2, g2.reshape(1, cout), be2.reshape(1, cout), w3,
      b3.reshape(1, c4), st3, g3.reshape(1, c4), be3.reshape(1, c4),
      yd, std, gd.reshape(1, c4), bed.reshape(1, c4))

    return jnp.transpose(out.reshape(ho, wo, n, c4), (2, 3, 0, 1))
